# baseline (device time: 99154 ns/iter reference)
import jax
import jax.numpy as jnp
from jax import lax
from jax.experimental import pallas as pl
from jax.experimental.pallas import tpu as pltpu

N_DEV = 8
B, Sq, Hq, Dh = 4, 256, 8, 128
D = Hq * Dh
CH = (B * Sq) // N_DEV
W = D + 128
SCALE = 0.08838834764831843
BF = jnp.bfloat16


def _unrank(v):
    return jnp.where(v < 4, v, 11 - v)


def kernel(x, Wq, Wo, K_ext, V_ext):
    x2 = x.reshape(B * Sq, D)

    def body(x_ref, wq_ref, wo_ref, k_ref, v_ref, out_ref,
             oacc_ref, lacc_ref, onb_ref, obp_ref, wqb_ref, wob_ref,
             sbuf_ref, commo_ref,
             sendo, recvo, sendo2, recvo2):
        my = lax.axis_index("i")
        r = _unrank(my)
        right = _unrank(lax.rem(r + 1, N_DEV))
        left = _unrank(lax.rem(r + N_DEV - 1, N_DEV))

        barrier_sem = pltpu.get_barrier_semaphore()
        for nbr in (left, right):
            pl.semaphore_signal(
                barrier_sem, inc=1,
                device_id=(nbr,), device_id_type=pl.DeviceIdType.MESH,
            )
        pl.semaphore_wait(barrier_sem, 2)

        wqb_ref[...] = wq_ref[...].astype(BF)
        wob_ref[...] = wo_ref[...].astype(BF)

        def compute_chunk(c):
            r0 = c * CH
            bb = lax.div(c, 2)
            qc = jnp.dot(
                x_ref[pl.ds(r0, CH), :].astype(BF), wqb_ref[...],
                preferred_element_type=jnp.float32,
            ) * SCALE
            for h in range(Hq):
                kbh = k_ref[bb, :, h, :]
                vbh = v_ref[bb, :, h, :]
                s = lax.dot_general(
                    qc[:, h * Dh:(h + 1) * Dh], kbh,
                    (((1,), (1,)), ((), ())),
                    preferred_element_type=jnp.float32,
                )
                p = jnp.exp(s)
                lacc_ref[pl.ds(r0, CH), h:h + 1] = lax.dot_general(
                    p, jnp.ones((Sq * 4, 1), jnp.float32),
                    (((1,), (0,)), ((), ())),
                    preferred_element_type=jnp.float32,
                )
                oacc_ref[pl.ds(r0, CH), h * Dh:(h + 1) * Dh] = (
                    lax.dot_general(
                        p, vbh, (((1,), (0,)), ((), ())),
                        preferred_element_type=jnp.float32,
                    )
                )

        compute_chunk(r)
        sbuf_ref[0, :, :D] = oacc_ref[pl.ds(r * CH, CH), :].astype(BF)
        sbuf_ref[0, :, D:D + Hq] = lacc_ref[pl.ds(r * CH, CH), :].astype(BF)
        for t in range(N_DEV - 1):
            slot = t % 2
            rc = lax.rem(r + N_DEV - t - 1, N_DEV)
            rdma_o = pltpu.make_async_remote_copy(
                src_ref=sbuf_ref.at[slot],
                dst_ref=commo_ref.at[slot],
                send_sem=sendo.at[t], recv_sem=recvo.at[t],
                device_id=(right,), device_id_type=pl.DeviceIdType.MESH,
            )
            rdma_o.start()
            compute_chunk(rc)
            rdma_o.wait()
            val = (
                oacc_ref[pl.ds(rc * CH, CH), :]
                + commo_ref[slot, :, :D].astype(jnp.float32)
            )
            lval = (
                lacc_ref[pl.ds(rc * CH, CH), :]
                + commo_ref[slot, :, D:D + Hq].astype(jnp.float32)
            )
            if t < N_DEV - 2:
                nslot = (t + 1) % 2
                sbuf_ref[nslot, :, :D] = val.astype(BF)
                sbuf_ref[nslot, :, D:D + Hq] = lval.astype(BF)
            else:
                for h in range(Hq):
                    onb_ref[:, h * Dh:(h + 1) * Dh] = (
                        val[:, h * Dh:(h + 1) * Dh] / lval[:, h:h + 1]
                    ).astype(BF)

        own = lax.rem(r + 1, N_DEV)
        o0 = own * CH
        proj = jnp.dot(
            onb_ref[...], wob_ref[...],
            preferred_element_type=jnp.float32,
        )
        obp_ref[pl.ds(o0, CH), :] = proj.astype(BF)
        out_ref[pl.ds(o0, CH), :] = proj

        rdmas = []
        for j in range(1, N_DEV):
            target = _unrank(lax.rem(r + j, N_DEV))
            rdma = pltpu.make_async_remote_copy(
                src_ref=obp_ref.at[pl.ds(o0, CH)],
                dst_ref=obp_ref.at[pl.ds(o0, CH)],
                send_sem=sendo2.at[j - 1], recv_sem=recvo2.at[j - 1],
                device_id=(target,), device_id_type=pl.DeviceIdType.MESH,
            )
            rdma.start()
            rdmas.append(rdma)
        for j, rdma in enumerate(rdmas, start=1):
            rdma.wait()
            cj = lax.rem(r + N_DEV + 1 - j, N_DEV)
            out_ref[pl.ds(cj * CH, CH), :] = (
                obp_ref[pl.ds(cj * CH, CH), :].astype(jnp.float32)
            )

    out = pl.pallas_call(
        body,
        out_shape=jax.ShapeDtypeStruct((B * Sq, D), jnp.float32),
        in_specs=[pl.BlockSpec(memory_space=pltpu.VMEM)] * 5,
        out_specs=pl.BlockSpec(memory_space=pltpu.VMEM),
        scratch_shapes=[
            pltpu.VMEM((B * Sq, D), jnp.float32),
            pltpu.VMEM((B * Sq, Hq), jnp.float32),
            pltpu.VMEM((CH, D), BF),
            pltpu.VMEM((B * Sq, D), BF),
            pltpu.VMEM((D, D), BF),
            pltpu.VMEM((D, D), BF),
            pltpu.VMEM((2, CH, W), BF),
            pltpu.VMEM((2, CH, W), BF),
            pltpu.SemaphoreType.DMA((N_DEV - 1,)),
            pltpu.SemaphoreType.DMA((N_DEV - 1,)),
            pltpu.SemaphoreType.DMA((N_DEV - 1,)),
            pltpu.SemaphoreType.DMA((N_DEV - 1,)),
        ],
        compiler_params=pltpu.CompilerParams(
            collective_id=0, vmem_limit_bytes=120 * 1024 * 1024
        ),
    )(x2, Wq, Wo, K_ext, V_ext)
    return out.reshape(B, Sq, D)


# device time: 91098 ns/iter; 1.0884x vs baseline; 1.0884x over previous
import jax
import jax.numpy as jnp
from jax import lax
from jax.experimental import pallas as pl
from jax.experimental.pallas import tpu as pltpu

N_DEV = 8
B, Sq, Hq, Dh = 4, 256, 8, 128
D = Hq * Dh
CH = (B * Sq) // N_DEV
W = D + 128
SCALE = 0.08838834764831843
BF = jnp.bfloat16


def _unrank(v):
    return jnp.where(v < 4, v, 11 - v)


def kernel(x, Wq, Wo, K_ext, V_ext):
    x2 = x.reshape(B * Sq, D)

    def body(x_ref, wq_ref, wo_ref, k_ref, v_ref, out_ref,
             oacc_ref, lacc_ref, onb_ref, obp_ref, wqb_ref, wob_ref,
             sbuf_ref, commo_ref,
             sendo, recvo, sendo2, recvo2):
        my = lax.axis_index("i")
        r = _unrank(my)
        right = _unrank(lax.rem(r + 1, N_DEV))
        left = _unrank(lax.rem(r + N_DEV - 1, N_DEV))

        barrier_sem = pltpu.get_barrier_semaphore()
        for nbr in (left, right):
            pl.semaphore_signal(
                barrier_sem, inc=1,
                device_id=(nbr,), device_id_type=pl.DeviceIdType.MESH,
            )
        pl.semaphore_wait(barrier_sem, 2)

        wqb_ref[...] = wq_ref[...].astype(BF)
        wob_ref[...] = wo_ref[...].astype(BF)

        def compute_chunk(c):
            r0 = c * CH
            bb = lax.div(c, 2)
            qc = jnp.dot(
                x_ref[pl.ds(r0, CH), :].astype(BF), wqb_ref[...],
                preferred_element_type=jnp.float32,
            ) * SCALE
            for h in range(Hq):
                kbh = k_ref[bb, :, h, :]
                vbh = v_ref[bb, :, h, :]
                s = lax.dot_general(
                    qc[:, h * Dh:(h + 1) * Dh], kbh,
                    (((1,), (1,)), ((), ())),
                    preferred_element_type=jnp.float32,
                )
                p = jnp.exp(s)
                lacc_ref[pl.ds(r0, CH), h:h + 1] = jnp.sum(
                    p, axis=1, keepdims=True
                )
                oacc_ref[pl.ds(r0, CH), h * Dh:(h + 1) * Dh] = (
                    lax.dot_general(
                        p, vbh, (((1,), (0,)), ((), ())),
                        preferred_element_type=jnp.float32,
                    )
                )

        compute_chunk(r)
        sbuf_ref[0, :, :D] = oacc_ref[pl.ds(r * CH, CH), :].astype(BF)
        sbuf_ref[0, :, D:D + Hq] = lacc_ref[pl.ds(r * CH, CH), :].astype(BF)
        for t in range(N_DEV - 1):
            slot = t % 2
            rc = lax.rem(r + N_DEV - t - 1, N_DEV)
            rdma_o = pltpu.make_async_remote_copy(
                src_ref=sbuf_ref.at[slot],
                dst_ref=commo_ref.at[slot],
                send_sem=sendo.at[t], recv_sem=recvo.at[t],
                device_id=(right,), device_id_type=pl.DeviceIdType.MESH,
            )
            rdma_o.start()
            compute_chunk(rc)
            rdma_o.wait()
            val = (
                oacc_ref[pl.ds(rc * CH, CH), :]
                + commo_ref[slot, :, :D].astype(jnp.float32)
            )
            lval = (
                lacc_ref[pl.ds(rc * CH, CH), :]
                + commo_ref[slot, :, D:D + Hq].astype(jnp.float32)
            )
            if t < N_DEV - 2:
                nslot = (t + 1) % 2
                sbuf_ref[nslot, :, :D] = val.astype(BF)
                sbuf_ref[nslot, :, D:D + Hq] = lval.astype(BF)
            else:
                for h in range(Hq):
                    onb_ref[:, h * Dh:(h + 1) * Dh] = (
                        val[:, h * Dh:(h + 1) * Dh] / lval[:, h:h + 1]
                    ).astype(BF)

        own = lax.rem(r + 1, N_DEV)
        o0 = own * CH
        proj = jnp.dot(
            onb_ref[...], wob_ref[...],
            preferred_element_type=jnp.float32,
        )
        obp_ref[pl.ds(o0, CH), :] = proj.astype(BF)
        out_ref[pl.ds(o0, CH), :] = proj

        rdmas = []
        for j in range(1, N_DEV):
            target = _unrank(lax.rem(r + j, N_DEV))
            rdma = pltpu.make_async_remote_copy(
                src_ref=obp_ref.at[pl.ds(o0, CH)],
                dst_ref=obp_ref.at[pl.ds(o0, CH)],
                send_sem=sendo2.at[j - 1], recv_sem=recvo2.at[j - 1],
                device_id=(target,), device_id_type=pl.DeviceIdType.MESH,
            )
            rdma.start()
            rdmas.append(rdma)
        for j, rdma in enumerate(rdmas, start=1):
            rdma.wait()
            cj = lax.rem(r + N_DEV + 1 - j, N_DEV)
            out_ref[pl.ds(cj * CH, CH), :] = (
                obp_ref[pl.ds(cj * CH, CH), :].astype(jnp.float32)
            )

    out = pl.pallas_call(
        body,
        out_shape=jax.ShapeDtypeStruct((B * Sq, D), jnp.float32),
        in_specs=[pl.BlockSpec(memory_space=pltpu.VMEM)] * 5,
        out_specs=pl.BlockSpec(memory_space=pltpu.VMEM),
        scratch_shapes=[
            pltpu.VMEM((B * Sq, D), jnp.float32),
            pltpu.VMEM((B * Sq, Hq), jnp.float32),
            pltpu.VMEM((CH, D), BF),
            pltpu.VMEM((B * Sq, D), BF),
            pltpu.VMEM((D, D), BF),
            pltpu.VMEM((D, D), BF),
            pltpu.VMEM((2, CH, W), BF),
            pltpu.VMEM((2, CH, W), BF),
            pltpu.SemaphoreType.DMA((N_DEV - 1,)),
            pltpu.SemaphoreType.DMA((N_DEV - 1,)),
            pltpu.SemaphoreType.DMA((N_DEV - 1,)),
            pltpu.SemaphoreType.DMA((N_DEV - 1,)),
        ],
        compiler_params=pltpu.CompilerParams(
            collective_id=0, vmem_limit_bytes=120 * 1024 * 1024
        ),
    )(x2, Wq, Wo, K_ext, V_ext)
    return out.reshape(B, Sq, D)
